# speedup vs baseline: 1.0228x; 1.0228x over previous
import jax
import jax.numpy as jnp
from jax.experimental import pallas as pl
from jax.experimental.pallas import tpu as pltpu

B, N, D, C, K = 4, 4096, 1024, 2, 256

NB_A = 1024
NB_B = 2048


def _router_body(z_ref, W1_ref, b1_ref, W2_ref, b2_ref, Wh_ref, bh_ref,
                 scores_ref, seg_ref):
    zb = z_ref[...]
    h = jnp.maximum(
        jax.lax.dot_general(zb, W1_ref[...], (((1,), (0,)), ((), ())),
                            preferred_element_type=jnp.float32) + b1_ref[...],
        0.0)
    s = jax.lax.dot_general(h, W2_ref[...], (((1,), (0,)), ((), ())),
                            preferred_element_type=jnp.float32) + b2_ref[...]
    scores_ref[...] = s
    seg_ref[...] = jax.lax.dot_general(
        zb, Wh_ref[...], (((1,), (0,)), ((), ())),
        preferred_element_type=jnp.float32) + bh_ref[...]


def _sortable_i32(bits):
    return jnp.where(bits < 0, bits ^ jnp.int32(0x7FFFFFFF), bits)


def _select_topk_mask(scores):
    kk = _sortable_i32(jax.lax.bitcast_convert_type(scores, jnp.int32))
    msb = jnp.int32(-2147483648)

    def step(i, prefix_u):
        bit = jnp.int32(1) << (jnp.int32(31) - i)
        cand_u = prefix_u | bit
        cand_s = cand_u ^ msb
        cnt = jnp.sum((kk >= cand_s).astype(jnp.int32), axis=1, keepdims=True)
        return jnp.where(cnt >= K, cand_u, prefix_u)

    prefix_u = jax.lax.fori_loop(0, 32, step, jnp.zeros((B, 1), jnp.int32))
    t_s = prefix_u ^ msb

    gt = kk > t_s
    eq = kk == t_s
    n_gt = jnp.sum(gt.astype(jnp.int32), axis=1, keepdims=True)
    need = K - n_gt

    col = jax.lax.broadcasted_iota(jnp.int32, (B, N), 1)
    fwd = jnp.int32(N) - col

    def step2(i, q):
        cand = q | (jnp.int32(1) << (jnp.int32(12) - i))
        cnt = jnp.sum((eq & (fwd >= cand)).astype(jnp.int32), axis=1,
                      keepdims=True)
        return jnp.where(cnt >= need, cand, q)

    q = jax.lax.fori_loop(0, 13, step2, jnp.zeros((B, 1), jnp.int32))
    sel_eq = eq & (fwd >= q)
    return gt | sel_eq


def _topk_clip_body(scores_ref, z_ref, Wh_ref, bh_ref,
                    w_ref, clip_logits_ref, acc_ref):
    i = pl.program_id(0)
    nblocks = pl.num_programs(0)

    @pl.when(i == 0)
    def _():
        mask = _select_topk_mask(scores_ref[...])
        w_ref[...] = jnp.where(mask, jnp.float32(1.0 / K), jnp.float32(0.0))
        acc_ref[...] = jnp.zeros_like(acc_ref)

    per_batch = N // NB_B
    b = i // per_batch
    col0 = (i % per_batch) * NB_B
    w_row = w_ref[pl.ds(b, 1), pl.ds(col0, NB_B)]
    acc_ref[pl.ds(b, 1), :] += jax.lax.dot_general(
        w_row, z_ref[...], (((1,), (0,)), ((), ())),
        preferred_element_type=jnp.float32)

    @pl.when(i == nblocks - 1)
    def _():
        clip_logits_ref[...] = jax.lax.dot_general(
            acc_ref[...], Wh_ref[...], (((1,), (0,)), ((), ())),
            preferred_element_type=jnp.float32) + bh_ref[...]


@jax.jit
def kernel(z, W1, b1, W2, b2, Wh, bh):
    z2d = z.reshape(B * N, D)

    scores2d, seg2d = pl.pallas_call(
        _router_body,
        grid=(B * N // NB_A,),
        in_specs=[
            pl.BlockSpec((NB_A, D), lambda i: (i, 0)),
            pl.BlockSpec((D, D), lambda i: (0, 0)),
            pl.BlockSpec((1, D), lambda i: (0, 0)),
            pl.BlockSpec((D, 1), lambda i: (0, 0)),
            pl.BlockSpec((1, 1), lambda i: (0, 0)),
            pl.BlockSpec((D, C), lambda i: (0, 0)),
            pl.BlockSpec((1, C), lambda i: (0, 0)),
        ],
        out_specs=[
            pl.BlockSpec((NB_A, 1), lambda i: (i, 0)),
            pl.BlockSpec((NB_A, C), lambda i: (i, 0)),
        ],
        out_shape=[
            jax.ShapeDtypeStruct((B * N, 1), jnp.float32),
            jax.ShapeDtypeStruct((B * N, C), jnp.float32),
        ],
    )(z2d, W1, b1.reshape(1, D), W2, b2.reshape(1, 1), Wh, bh.reshape(1, C))

    scores = scores2d.reshape(B, N)

    w, clip_logits = pl.pallas_call(
        _topk_clip_body,
        grid=(B * N // NB_B,),
        in_specs=[
            pl.BlockSpec((B, N), lambda i: (0, 0)),
            pl.BlockSpec((NB_B, D), lambda i: (i, 0)),
            pl.BlockSpec((D, C), lambda i: (0, 0)),
            pl.BlockSpec((1, C), lambda i: (0, 0)),
        ],
        out_specs=[
            pl.BlockSpec((B, N), lambda i: (0, 0)),
            pl.BlockSpec((B, C), lambda i: (0, 0)),
        ],
        out_shape=[
            jax.ShapeDtypeStruct((B, N), jnp.float32),
            jax.ShapeDtypeStruct((B, C), jnp.float32),
        ],
        scratch_shapes=[pltpu.VMEM((B, D), jnp.float32)],
    )(scores, z2d, Wh, bh.reshape(1, C))

    return clip_logits, seg2d.reshape(B, N, C), w


# speedup vs baseline: 1.0838x; 1.0596x over previous
import jax
import jax.numpy as jnp
from jax.experimental import pallas as pl
from jax.experimental.pallas import tpu as pltpu

B, N, D, C, K = 4, 4096, 1024, 2, 256

NB_A = 1024
NB_B = 2048


def _router_body(z_ref, Wc_ref, b1_ref, W2t_ref, b2_ref, bh_ref,
                 scores_ref, seg_ref):
    zb = z_ref[...]
    combined = jax.lax.dot_general(zb, Wc_ref[...], (((1,), (0,)), ((), ())),
                                   preferred_element_type=jnp.float32)
    h = jnp.maximum(combined[:, :D] + b1_ref[...], 0.0)
    seg_ref[...] = combined[:, D:D + C] + bh_ref[...]
    s = jnp.sum(h * W2t_ref[...], axis=1, keepdims=True) + b2_ref[...]
    scores_ref[...] = s


def _sortable_i32(bits):
    return jnp.where(bits < 0, bits ^ jnp.int32(0x7FFFFFFF), bits)


def _select_topk_mask(scores):
    kk = _sortable_i32(jax.lax.bitcast_convert_type(scores, jnp.int32))
    msb = jnp.int32(-2147483648)

    def step(i, prefix_u):
        bit = jnp.int32(1) << (jnp.int32(31) - i)
        cand_u = prefix_u | bit
        cand_s = cand_u ^ msb
        cnt = jnp.sum((kk >= cand_s).astype(jnp.int32), axis=1, keepdims=True)
        return jnp.where(cnt >= K, cand_u, prefix_u)

    prefix_u = jax.lax.fori_loop(0, 32, step, jnp.zeros((B, 1), jnp.int32))
    t_s = prefix_u ^ msb

    gt = kk > t_s
    eq = kk == t_s
    n_gt = jnp.sum(gt.astype(jnp.int32), axis=1, keepdims=True)
    need = K - n_gt

    col = jax.lax.broadcasted_iota(jnp.int32, (B, N), 1)
    fwd = jnp.int32(N) - col

    def step2(i, q):
        cand = q | (jnp.int32(1) << (jnp.int32(12) - i))
        cnt = jnp.sum((eq & (fwd >= cand)).astype(jnp.int32), axis=1,
                      keepdims=True)
        return jnp.where(cnt >= need, cand, q)

    q = jax.lax.fori_loop(0, 13, step2, jnp.zeros((B, 1), jnp.int32))
    sel_eq = eq & (fwd >= q)
    return gt | sel_eq


def _topk_clip_body(scores_ref, z_ref, Wh_ref, bh_ref,
                    w_ref, clip_logits_ref, acc_ref):
    i = pl.program_id(0)
    nblocks = pl.num_programs(0)

    @pl.when(i == 0)
    def _():
        mask = _select_topk_mask(scores_ref[...])
        w_ref[...] = jnp.where(mask, jnp.float32(1.0 / K), jnp.float32(0.0))
        acc_ref[...] = jnp.zeros_like(acc_ref)

    per_batch = N // NB_B
    b = i // per_batch
    col0 = (i % per_batch) * NB_B
    w_row = w_ref[pl.ds(b, 1), pl.ds(col0, NB_B)]
    acc_ref[pl.ds(b, 1), :] += jax.lax.dot_general(
        w_row, z_ref[...], (((1,), (0,)), ((), ())),
        preferred_element_type=jnp.float32)

    @pl.when(i == nblocks - 1)
    def _():
        clip_logits_ref[...] = jax.lax.dot_general(
            acc_ref[...], Wh_ref[...], (((1,), (0,)), ((), ())),
            preferred_element_type=jnp.float32) + bh_ref[...]


@jax.jit
def kernel(z, W1, b1, W2, b2, Wh, bh):
    z2d = z.reshape(B * N, D)
    Wc = jnp.concatenate([W1, Wh], axis=1)

    scores2d, seg2d = pl.pallas_call(
        _router_body,
        grid=(B * N // NB_A,),
        in_specs=[
            pl.BlockSpec((NB_A, D), lambda i: (i, 0)),
            pl.BlockSpec((D, D + C), lambda i: (0, 0)),
            pl.BlockSpec((1, D), lambda i: (0, 0)),
            pl.BlockSpec((1, D), lambda i: (0, 0)),
            pl.BlockSpec((1, 1), lambda i: (0, 0)),
            pl.BlockSpec((1, C), lambda i: (0, 0)),
        ],
        out_specs=[
            pl.BlockSpec((NB_A, 1), lambda i: (i, 0)),
            pl.BlockSpec((NB_A, C), lambda i: (i, 0)),
        ],
        out_shape=[
            jax.ShapeDtypeStruct((B * N, 1), jnp.float32),
            jax.ShapeDtypeStruct((B * N, C), jnp.float32),
        ],
    )(z2d, Wc, b1.reshape(1, D), W2.reshape(1, D), b2.reshape(1, 1),
      bh.reshape(1, C))

    scores = scores2d.reshape(B, N)

    w, clip_logits = pl.pallas_call(
        _topk_clip_body,
        grid=(B * N // NB_B,),
        in_specs=[
            pl.BlockSpec((B, N), lambda i: (0, 0)),
            pl.BlockSpec((NB_B, D), lambda i: (i, 0)),
            pl.BlockSpec((D, C), lambda i: (0, 0)),
            pl.BlockSpec((1, C), lambda i: (0, 0)),
        ],
        out_specs=[
            pl.BlockSpec((B, N), lambda i: (0, 0)),
            pl.BlockSpec((B, C), lambda i: (0, 0)),
        ],
        out_shape=[
            jax.ShapeDtypeStruct((B, N), jnp.float32),
            jax.ShapeDtypeStruct((B, C), jnp.float32),
        ],
        scratch_shapes=[pltpu.VMEM((B, D), jnp.float32)],
    )(scores, z2d, Wh, bh.reshape(1, C))

    return clip_logits, seg2d.reshape(B, N, C), w


# speedup vs baseline: 1.1455x; 1.0569x over previous
import jax
import jax.numpy as jnp
from jax.experimental import pallas as pl
from jax.experimental.pallas import tpu as pltpu

B, N, D, C, K = 4, 4096, 1024, 2, 256

NB_A = 1024
NB_B = 2048


def _router_body(z_ref, Wc_ref, b1_ref, W2_ref, b2_ref, bh_ref,
                 scores_ref, seg_ref):
    zb = z_ref[...]
    combined = jax.lax.dot_general(zb, Wc_ref[...], (((1,), (0,)), ((), ())),
                                   preferred_element_type=jnp.float32)
    h = jnp.maximum(combined[:, :D] + b1_ref[...], 0.0)
    seg_ref[...] = combined[:, D:D + C] + bh_ref[...]
    s = jax.lax.dot_general(h, W2_ref[...], (((1,), (0,)), ((), ())),
                            preferred_element_type=jnp.float32) + b2_ref[...]
    scores_ref[...] = s


def _sortable_i32(bits):
    return jnp.where(bits < 0, bits ^ jnp.int32(0x7FFFFFFF), bits)


def _select_topk_mask(scores):
    kk = _sortable_i32(jax.lax.bitcast_convert_type(scores, jnp.int32))
    msb = jnp.int32(-2147483648)

    def step(i, prefix_u):
        bit = jnp.int32(1) << (jnp.int32(31) - i)
        cand_u = prefix_u | bit
        cand_s = cand_u ^ msb
        cnt = jnp.sum((kk >= cand_s).astype(jnp.int32), axis=1, keepdims=True)
        return jnp.where(cnt >= K, cand_u, prefix_u)

    prefix_u = jax.lax.fori_loop(0, 32, step, jnp.zeros((B, 1), jnp.int32))
    t_s = prefix_u ^ msb

    gt = kk > t_s
    eq = kk == t_s
    n_gt = jnp.sum(gt.astype(jnp.int32), axis=1, keepdims=True)
    need = K - n_gt

    col = jax.lax.broadcasted_iota(jnp.int32, (B, N), 1)
    fwd = jnp.int32(N) - col

    def step2(i, q):
        cand = q | (jnp.int32(1) << (jnp.int32(12) - i))
        cnt = jnp.sum((eq & (fwd >= cand)).astype(jnp.int32), axis=1,
                      keepdims=True)
        return jnp.where(cnt >= need, cand, q)

    q = jax.lax.fori_loop(0, 13, step2, jnp.zeros((B, 1), jnp.int32))
    sel_eq = eq & (fwd >= q)
    return gt | sel_eq


def _topk_clip_body(scores_ref, z_ref, Wh_ref, bh_ref,
                    w_ref, clip_logits_ref, acc_ref):
    i = pl.program_id(0)
    nblocks = pl.num_programs(0)

    @pl.when(i == 0)
    def _():
        mask = _select_topk_mask(scores_ref[...])
        w_ref[...] = jnp.where(mask, jnp.float32(1.0 / K), jnp.float32(0.0))
        acc_ref[...] = jnp.zeros_like(acc_ref)

    per_batch = N // NB_B
    b = i // per_batch
    col0 = (i % per_batch) * NB_B
    w_row = w_ref[pl.ds(b, 1), pl.ds(col0, NB_B)]
    acc_ref[pl.ds(b, 1), :] += jax.lax.dot_general(
        w_row, z_ref[...], (((1,), (0,)), ((), ())),
        preferred_element_type=jnp.float32)

    @pl.when(i == nblocks - 1)
    def _():
        clip_logits_ref[...] = jax.lax.dot_general(
            acc_ref[...], Wh_ref[...], (((1,), (0,)), ((), ())),
            preferred_element_type=jnp.float32) + bh_ref[...]


@jax.jit
def kernel(z, W1, b1, W2, b2, Wh, bh):
    z2d = z.reshape(B * N, D)
    Wc = jnp.concatenate([W1, Wh], axis=1)

    scores2d, seg2d = pl.pallas_call(
        _router_body,
        grid=(B * N // NB_A,),
        in_specs=[
            pl.BlockSpec((NB_A, D), lambda i: (i, 0)),
            pl.BlockSpec((D, D + C), lambda i: (0, 0)),
            pl.BlockSpec((1, D), lambda i: (0, 0)),
            pl.BlockSpec((D, 1), lambda i: (0, 0)),
            pl.BlockSpec((1, 1), lambda i: (0, 0)),
            pl.BlockSpec((1, C), lambda i: (0, 0)),
        ],
        out_specs=[
            pl.BlockSpec((NB_A, 1), lambda i: (i, 0)),
            pl.BlockSpec((NB_A, C), lambda i: (i, 0)),
        ],
        out_shape=[
            jax.ShapeDtypeStruct((B * N, 1), jnp.float32),
            jax.ShapeDtypeStruct((B * N, C), jnp.float32),
        ],
    )(z2d, Wc, b1.reshape(1, D), W2, b2.reshape(1, 1), bh.reshape(1, C))

    scores = scores2d.reshape(B, N)

    w, clip_logits = pl.pallas_call(
        _topk_clip_body,
        grid=(B * N // NB_B,),
        in_specs=[
            pl.BlockSpec((B, N), lambda i: (0, 0)),
            pl.BlockSpec((NB_B, D), lambda i: (i, 0)),
            pl.BlockSpec((D, C), lambda i: (0, 0)),
            pl.BlockSpec((1, C), lambda i: (0, 0)),
        ],
        out_specs=[
            pl.BlockSpec((B, N), lambda i: (0, 0)),
            pl.BlockSpec((B, C), lambda i: (0, 0)),
        ],
        out_shape=[
            jax.ShapeDtypeStruct((B, N), jnp.float32),
            jax.ShapeDtypeStruct((B, C), jnp.float32),
        ],
        scratch_shapes=[pltpu.VMEM((B, D), jnp.float32)],
    )(scores, z2d, Wh, bh.reshape(1, C))

    return clip_logits, seg2d.reshape(B, N, C), w
